# Initial kernel scaffold; baseline (speedup 1.0000x reference)
import jax
import jax.numpy as jnp
from jax.experimental import pallas as pl


def kernel(preds, input, height, width, warp_matrix):
    raise NotImplementedError("write your pallas kernel here")


# speedup vs baseline: 11.9924x; 11.9924x over previous
import jax
import jax.numpy as jnp
from jax import lax
from jax.experimental import pallas as pl
from jax.experimental.pallas import tpu as pltpu

_R, _C = 8, 2500
_NCLS = 80
_CONF = 0.35
_IOU = 0.6
_MAXDET = 100
_NEGINF = float("-inf")


def _sig(x):
    return 1.0 / (1.0 + jnp.exp(-x))


def _nms_body(pt_ref, hw_ref, wm_ref, out_ref):
    height = hw_ref[0]
    width = hw_ref[1]

    cx = _sig(pt_ref[0]) * width
    cy = _sig(pt_ref[1]) * height
    w = _sig(pt_ref[2]) * width * 0.3
    h = _sig(pt_ref[3]) * height * 0.3
    x1 = cx - w * 0.5
    y1 = cy - h * 0.5
    x2 = cx + w * 0.5
    y2 = cy + h * 0.5

    def cls_step(k, carry):
        bs, bl = carry
        c = pt_ref[k]
        gt = c > bs
        bs = jnp.where(gt, c, bs)
        bl = jnp.where(gt, (k - 4).astype(jnp.float32), bl)
        return bs, bl

    best_logit, labf = lax.fori_loop(
        5, 4 + _NCLS, cls_step,
        (pt_ref[4], jnp.zeros((_R, _C), jnp.float32)))
    scores = _sig(best_logit)

    s0 = jnp.where(scores > _CONF, scores, _NEGINF)

    mc = jnp.maximum(jnp.maximum(jnp.max(x1), jnp.max(y1)),
                     jnp.maximum(jnp.max(x2), jnp.max(y2)))
    off = labf * (mc + 1.0)
    nx1 = x1 + off
    ny1 = y1 + off
    nx2 = x2 + off
    ny2 = y2 + off
    areas = jnp.maximum(nx2 - nx1, 0.0) * jnp.maximum(ny2 - ny1, 0.0)

    idx = (lax.broadcasted_iota(jnp.int32, (_R, _C), 0) * _C
           + lax.broadcasted_iota(jnp.int32, (_R, _C), 1))
    big = jnp.int32(2 ** 30)

    a, b, c_ = wm_ref[0, 0], wm_ref[0, 1], wm_ref[0, 2]
    d, e, f = wm_ref[1, 0], wm_ref[1, 1], wm_ref[1, 2]
    g, hh, i_ = wm_ref[2, 0], wm_ref[2, 1], wm_ref[2, 2]
    det = a * (e * i_ - f * hh) - b * (d * i_ - f * g) + c_ * (d * hh - e * g)
    rdet = 1.0 / det
    i00 = (e * i_ - f * hh) * rdet
    i01 = (c_ * hh - b * i_) * rdet
    i02 = (b * f - c_ * e) * rdet
    i10 = (f * g - d * i_) * rdet
    i11 = (a * i_ - c_ * g) * rdet
    i12 = (c_ * d - a * f) * rdet
    i20 = (d * hh - e * g) * rdet
    i21 = (b * g - a * hh) * rdet
    i22 = (a * e - b * d) * rdet

    lane = lax.broadcasted_iota(jnp.int32, (1, 128), 1)

    def nms_step(i, s):
        m = jnp.max(s)
        idxs = jnp.where(s == m, idx, big)
        best = jnp.min(idxs)
        one = idxs == best
        valid = m != _NEGINF

        def ext(v):
            return jnp.sum(jnp.where(one, v, 0.0))

        bx1 = ext(nx1)
        by1 = ext(ny1)
        bx2 = ext(nx2)
        by2 = ext(ny2)
        bl = ext(labf)

        a1 = jnp.maximum(bx2 - bx1, 0.0) * jnp.maximum(by2 - by1, 0.0)
        iw = jnp.maximum(jnp.minimum(bx2, nx2) - jnp.maximum(bx1, nx1), 0.0)
        ih = jnp.maximum(jnp.minimum(by2, ny2) - jnp.maximum(by1, ny1), 0.0)
        inter = iw * ih
        iou = inter / (a1 + areas - inter + 1e-7)
        s = jnp.where(iou > _IOU, _NEGINF, s)
        s = jnp.where(one, _NEGINF, s)

        ob = bl * (mc + 1.0)
        ox1 = bx1 - ob
        oy1 = by1 - ob
        ox2 = bx2 - ob
        oy2 = by2 - ob

        def warp(x, y):
            den = i20 * x + i21 * y + i22 + 1e-9
            return ((i00 * x + i01 * y + i02) / den,
                    (i10 * x + i11 * y + i12) / den)

        xa, ya = warp(ox1, oy1)
        xb, yb = warp(ox2, oy1)
        xc, yc = warp(ox1, oy2)
        xd, yd = warp(ox2, oy2)
        wx1 = jnp.clip(jnp.minimum(jnp.minimum(xa, xb), jnp.minimum(xc, xd)), 0.0, width)
        wy1 = jnp.clip(jnp.minimum(jnp.minimum(ya, yb), jnp.minimum(yc, yd)), 0.0, height)
        wx2 = jnp.clip(jnp.maximum(jnp.maximum(xa, xb), jnp.maximum(xc, xd)), 0.0, width)
        wy2 = jnp.clip(jnp.maximum(jnp.maximum(ya, yb), jnp.maximum(yc, yd)), 0.0, height)

        row = jnp.where(lane == 0, wx1,
              jnp.where(lane == 1, wy1,
              jnp.where(lane == 2, wx2,
              jnp.where(lane == 3, wy2,
              jnp.where(lane == 4, m,
              jnp.where(lane == 5, bl, 0.0))))))
        row = jnp.where(valid, row, 0.0)
        out_ref[pl.ds(i, 1), :] = row
        return s

    lax.fori_loop(0, _MAXDET, nms_step, s0)


def kernel(preds, input, height, width, warp_matrix):
    del input
    pt = preds.astype(jnp.float16).astype(jnp.float32)
    pt = pt.reshape(_R, _C, 4 + _NCLS).transpose(2, 0, 1)
    hw = jnp.stack([height, width])
    dets = pl.pallas_call(
        _nms_body,
        out_shape=jax.ShapeDtypeStruct((_MAXDET, 128), jnp.float32),
        in_specs=[
            pl.BlockSpec(memory_space=pltpu.VMEM),
            pl.BlockSpec(memory_space=pltpu.SMEM),
            pl.BlockSpec(memory_space=pltpu.SMEM),
        ],
        out_specs=pl.BlockSpec(memory_space=pltpu.VMEM),
    )(pt, hw, warp_matrix)
    return dets[:, :6]


# speedup vs baseline: 12.7632x; 1.0643x over previous
import jax
import jax.numpy as jnp
from jax import lax
from jax.experimental import pallas as pl
from jax.experimental.pallas import tpu as pltpu

_R, _C = 8, 2500
_NCLS = 80
_CONF = 0.35
_IOU = 0.6
_MAXDET = 100
_NEGINF = float("-inf")


def _sig(x):
    return 1.0 / (1.0 + jnp.exp(-x))


def _nms_body(pt_ref, hw_ref, wm_ref, out_ref):
    height = hw_ref[0]
    width = hw_ref[1]

    cx = _sig((pt_ref[0])) * width
    cy = _sig((pt_ref[1])) * height
    w = _sig((pt_ref[2])) * width * 0.3
    h = _sig((pt_ref[3])) * height * 0.3
    x1 = cx - w * 0.5
    y1 = cy - h * 0.5
    x2 = cx + w * 0.5
    y2 = cy + h * 0.5

    def cls_step(k, carry):
        bs, bl = carry
        c = pt_ref[k]
        gt = c > bs
        bs = jnp.where(gt, c, bs)
        bl = jnp.where(gt, (k - 4).astype(jnp.float32), bl)
        return bs, bl

    best_logit, labf = lax.fori_loop(
        5, 4 + _NCLS, cls_step,
        (pt_ref[4], jnp.zeros((_R, _C), jnp.float32)))
    scores = _sig(best_logit)

    s0 = jnp.where(scores > _CONF, scores, _NEGINF)

    mc = jnp.maximum(jnp.maximum(jnp.max(x1), jnp.max(y1)),
                     jnp.maximum(jnp.max(x2), jnp.max(y2)))
    mc1 = mc + 1.0
    off = labf * mc1
    nx1 = x1 + off
    ny1 = y1 + off
    nx2 = x2 + off
    ny2 = y2 + off
    areas = jnp.maximum(nx2 - nx1, 0.0) * jnp.maximum(ny2 - ny1, 0.0)

    idx = (lax.broadcasted_iota(jnp.int32, (_R, _C), 0) * _C
           + lax.broadcasted_iota(jnp.int32, (_R, _C), 1))
    big = jnp.int32(2 ** 30)
    lane = lax.broadcasted_iota(jnp.int32, (1, 128), 1)

    def nms_step(i, s):
        m = jnp.max(s)
        idxs = jnp.where(s == m, idx, big)
        best = jnp.min(idxs)
        one = idxs == best

        def ext(v):
            return jnp.sum(jnp.where(one, v, 0.0))

        bx1 = ext(nx1)
        by1 = ext(ny1)
        bx2 = ext(nx2)
        by2 = ext(ny2)

        a1 = jnp.maximum(bx2 - bx1, 0.0) * jnp.maximum(by2 - by1, 0.0)
        iw = jnp.maximum(jnp.minimum(bx2, nx2) - jnp.maximum(bx1, nx1), 0.0)
        ih = jnp.maximum(jnp.minimum(by2, ny2) - jnp.maximum(by1, ny1), 0.0)
        inter = iw * ih
        sup = inter > _IOU * (a1 + areas - inter + 1e-7)
        s = jnp.where(sup | one, _NEGINF, s)

        row = jnp.where(lane == 0, bx1,
              jnp.where(lane == 1, by1,
              jnp.where(lane == 2, bx2,
              jnp.where(lane == 3, by2,
              jnp.where(lane == 4, m, 0.0)))))
        out_ref[pl.ds(i, 1), :] = row
        return s

    lax.fori_loop(0, _MAXDET, nms_step, s0, unroll=2)

    sel = out_ref[...]
    bx1 = sel[:, 0:1]
    by1 = sel[:, 1:2]
    bx2 = sel[:, 2:3]
    by2 = sel[:, 3:4]
    mcol = sel[:, 4:5]
    valid = mcol != _NEGINF

    bl = jnp.floor(bx2 / mc1 + 0.001)
    ob = bl * mc1
    ox1 = bx1 - ob
    oy1 = by1 - ob
    ox2 = bx2 - ob
    oy2 = by2 - ob

    a, b, c_ = wm_ref[0, 0], wm_ref[0, 1], wm_ref[0, 2]
    d, e, f = wm_ref[1, 0], wm_ref[1, 1], wm_ref[1, 2]
    g, hh, i_ = wm_ref[2, 0], wm_ref[2, 1], wm_ref[2, 2]
    det = a * (e * i_ - f * hh) - b * (d * i_ - f * g) + c_ * (d * hh - e * g)
    rdet = 1.0 / det
    i00 = (e * i_ - f * hh) * rdet
    i01 = (c_ * hh - b * i_) * rdet
    i02 = (b * f - c_ * e) * rdet
    i10 = (f * g - d * i_) * rdet
    i11 = (a * i_ - c_ * g) * rdet
    i12 = (c_ * d - a * f) * rdet
    i20 = (d * hh - e * g) * rdet
    i21 = (b * g - a * hh) * rdet
    i22 = (a * e - b * d) * rdet

    def warp(x, y):
        den = i20 * x + i21 * y + i22 + 1e-9
        return ((i00 * x + i01 * y + i02) / den,
                (i10 * x + i11 * y + i12) / den)

    xa, ya = warp(ox1, oy1)
    xb, yb = warp(ox2, oy1)
    xc, yc = warp(ox1, oy2)
    xd, yd = warp(ox2, oy2)
    wx1 = jnp.clip(jnp.minimum(jnp.minimum(xa, xb), jnp.minimum(xc, xd)), 0.0, width)
    wy1 = jnp.clip(jnp.minimum(jnp.minimum(ya, yb), jnp.minimum(yc, yd)), 0.0, height)
    wx2 = jnp.clip(jnp.maximum(jnp.maximum(xa, xb), jnp.maximum(xc, xd)), 0.0, width)
    wy2 = jnp.clip(jnp.maximum(jnp.maximum(ya, yb), jnp.maximum(yc, yd)), 0.0, height)

    dets = jnp.where(lane == 0, wx1,
           jnp.where(lane == 1, wy1,
           jnp.where(lane == 2, wx2,
           jnp.where(lane == 3, wy2,
           jnp.where(lane == 4, mcol,
           jnp.where(lane == 5, bl, 0.0))))))
    dets = jnp.where(valid, dets, 0.0)
    out_ref[...] = dets


def kernel(preds, input, height, width, warp_matrix):
    del input
    pt = preds.astype(jnp.float16).astype(jnp.float32)
    pt = pt.reshape(_R, _C, 4 + _NCLS).transpose(2, 0, 1)
    hw = jnp.stack([height, width])
    dets = pl.pallas_call(
        _nms_body,
        out_shape=jax.ShapeDtypeStruct((_MAXDET, 128), jnp.float32),
        in_specs=[
            pl.BlockSpec(memory_space=pltpu.VMEM),
            pl.BlockSpec(memory_space=pltpu.SMEM),
            pl.BlockSpec(memory_space=pltpu.SMEM),
        ],
        out_specs=pl.BlockSpec(memory_space=pltpu.VMEM),
    )(pt, hw, warp_matrix)
    return dets[:, :6]
